# baseline (speedup 1.0000x reference)
import functools

import jax
import jax.numpy as jnp
from jax import lax
from jax.experimental import pallas as pl
from jax.experimental.pallas import tpu as pltpu
from jax.experimental.pallas import tpu_sc as plsc

L = 16
NWORKERS = 32
NACC = 4
TN_PER_CHUNK = 2
T_SC = 8
TC_ROWS = 128


def _row_max_argmax(load, nvec, iota):
    vm = []
    vi = []
    for a in range(NACC):
        vm.append(load(a))
        vi.append(jnp.full((L,), a, jnp.int32))
    for j in range(NACC, nvec):
        a = j % NACC
        v = load(j)
        pred = v > vm[a]
        vm[a] = jnp.where(pred, v, vm[a])
        vi[a] = jnp.where(pred, j, vi[a])

    def merge(m1, i1, m2, i2):
        take = (m2 > m1) | ((m2 == m1) & (i2 < i1))
        return jnp.where(take, m2, m1), jnp.where(take, i2, i1)

    m01, i01 = merge(vm[0], vi[0], vm[1], vi[1])
    m23, i23 = merge(vm[2], vi[2], vm[3], vi[3])
    m, i = merge(m01, i01, m23, i23)

    mval = jnp.max(m)
    eidx = i * L + iota
    sel = jnp.where(m == mval, eidx, jnp.int32(2147483647))
    return mval, jnp.min(sel)


def _make_sc_kernel(n, k, wx, wy, n_sc):
    twx = wx // 128
    twy = wy // 128
    tn_per_w = (n_sc // 8) // NWORKERS
    cc_per_k = tn_per_w // TN_PER_CHUNK
    nchunks = k * cc_per_k
    nloc = n_sc // NWORKERS
    rows_per_w = nloc * k
    mesh = plsc.VectorSubcoreMesh(core_axis_name="c", subcore_axis_name="s")

    @functools.partial(
        pl.kernel,
        mesh=mesh,
        compiler_params=pltpu.CompilerParams(needs_layout_passes=False),
        out_type=[
            jax.ShapeDtypeStruct((k * n_sc * 2,), jnp.float32),
            jax.ShapeDtypeStruct((k * n_sc,), jnp.float32),
        ],
        scratch_types=[
            pltpu.VMEM((TN_PER_CHUNK, twx, 8, 128), jnp.float32),
            pltpu.VMEM((TN_PER_CHUNK, twx, 8, 128), jnp.float32),
            pltpu.VMEM((TN_PER_CHUNK, twy, 8, 128), jnp.float32),
            pltpu.VMEM((TN_PER_CHUNK, twy, 8, 128), jnp.float32),
            pltpu.VMEM((rows_per_w * 2,), jnp.float32),
            pltpu.VMEM((rows_per_w,), jnp.float32),
            pltpu.SemaphoreType.DMA,
            pltpu.SemaphoreType.DMA,
            pltpu.SemaphoreType.DMA,
            pltpu.SemaphoreType.DMA,
        ],
    )
    def sc_kernel(x_hbm, y_hbm, kp_hbm, sc_hbm,
                  xb0, xb1, yb0, yb1, kp_v, sc_v,
                  sx0, sx1, sy0, sy1):
        wid = lax.axis_index("s") * 2 + lax.axis_index("c")
        tn0 = wid * tn_per_w
        xbufs = (xb0, xb1)
        ybufs = (yb0, yb1)
        xsems = (sx0, sx1)
        ysems = (sy0, sy1)

        def srcs(g):
            kk = g // cc_per_k
            cc = g % cc_per_k
            t_lo = tn0 + cc * TN_PER_CHUNK
            return (x_hbm.at[kk, pl.ds(t_lo, TN_PER_CHUNK)],
                    y_hbm.at[kk, pl.ds(t_lo, TN_PER_CHUNK)])

        for b in range(2):
            xs, ys = srcs(b)
            pltpu.make_async_copy(xs, xbufs[b], xsems[b]).start()
            pltpu.make_async_copy(ys, ybufs[b], ysems[b]).start()

        def do_chunk(g, b):
            xs, ys = srcs(g)
            pltpu.make_async_copy(xs, xbufs[b], xsems[b]).wait()
            pltpu.make_async_copy(ys, ybufs[b], ysems[b]).wait()
            kk = g // cc_per_k
            cc = g % cc_per_k

            iota = lax.iota(jnp.int32, L)
            zero = jnp.zeros((L,), jnp.float32)

            for h in range(TN_PER_CHUNK * 8 // L):

                def row_body(rr, carry, h=h):
                    valv, fxv, fyv = carry
                    s = rr // 8
                    r = rr % 8

                    def xload(j):
                        return xbufs[b][s, j // 8, r, pl.ds((j % 8) * L, L)]

                    def yload(j):
                        return ybufs[b][s, j // 8, r, pl.ds((j % 8) * L, L)]

                    xm, xi = _row_max_argmax(xload, (twx * 128) // L, iota)
                    ym, yi = _row_max_argmax(yload, (twy * 128) // L, iota)
                    val = jnp.minimum(xm, ym)
                    neg = val <= jnp.float32(0.0)
                    fx = jnp.where(neg, jnp.float32(-1.0),
                                   xi.astype(jnp.float32)) * jnp.float32(0.5)
                    fy = jnp.where(neg, jnp.float32(-1.0),
                                   yi.astype(jnp.float32)) * jnp.float32(0.5)
                    lanehit = iota == (rr - h * L if h else rr)
                    return (jnp.where(lanehit, val, valv),
                            jnp.where(lanehit, fx, fxv),
                            jnp.where(lanehit, fy, fyv))

                valv, fxv, fyv = lax.fori_loop(
                    h * L, (h + 1) * L, row_body, (zero, zero, zero))
                m0 = TN_PER_CHUNK * 8 * cc + h * L
                plsc.store_scatter(sc_v, [(m0 * k + kk) + k * iota], valv)
                kp_v[pl.ds(kk * 2 * nloc + m0, L)] = fxv
                kp_v[pl.ds(kk * 2 * nloc + nloc + m0, L)] = fyv

            @pl.when(g + 2 < nchunks)
            def _():
                xs2, ys2 = srcs(g + 2)
                pltpu.make_async_copy(xs2, xbufs[b], xsems[b]).start()
                pltpu.make_async_copy(ys2, ybufs[b], ysems[b]).start()

        def pair_body(i, _):
            do_chunk(2 * i, 0)
            do_chunk(2 * i + 1, 1)
            return 0

        lax.fori_loop(0, nchunks // 2, pair_body, 0)

        kp_copies = []
        for kk_s in range(k):
            for c in range(2):
                src = kp_v.at[pl.ds(kk_s * 2 * nloc + c * nloc, nloc)]
                dst = kp_hbm.at[pl.ds(kk_s * 2 * n_sc
                                      + (wid * nloc // 128) * 256
                                      + c * 128 + (wid * nloc) % 128, nloc)]
                kp_copies.append(pltpu.make_async_copy(src, dst, sx0))
        for cp in kp_copies:
            cp.start()
        for cp in kp_copies:
            cp.wait()
        pltpu.sync_copy(sc_v, sc_hbm.at[pl.ds(wid * rows_per_w, rows_per_w)])

    return sc_kernel


def _make_tc_kernel(n, k, wx, wy, t0, ntiles):
    twx = wx // 128
    twy = wy // 128
    n_tc = ntiles * 128
    steps = n_tc // TC_ROWS
    su = TC_ROWS // 8

    def body(x_ref, y_ref, kp_ref, sc_ref):
        big = jnp.int32(2147483647)
        q = lax.rem(pl.program_id(1), 128 // TC_ROWS)

        def stats(b):
            shape = b.shape
            ei = (lax.broadcasted_iota(jnp.int32, shape, 1) * 128
                  + lax.broadcasted_iota(jnp.int32, shape, 3))
            m = jnp.max(b, axis=(1, 3))
            sel = jnp.where(b == m[:, None, :, None], ei, big)
            return m, jnp.min(sel, axis=(1, 3))

        xm, xi = stats(x_ref[0])
        ym, yi = stats(y_ref[0])
        val = jnp.minimum(xm, ym)
        neg = val <= jnp.float32(0.0)
        fx = jnp.where(neg, jnp.float32(-1.0),
                       xi.astype(jnp.float32)) * jnp.float32(0.5)
        fy = jnp.where(neg, jnp.float32(-1.0),
                       yi.astype(jnp.float32)) * jnp.float32(0.5)
        tt = pl.program_id(1) // (128 // TC_ROWS)
        sc_ref[0, tt, pl.ds(q * TC_ROWS, TC_ROWS)] = val.reshape(TC_ROWS)
        kp_ref[0, 0, 0, pl.ds(q * TC_ROWS, TC_ROWS)] = fx.reshape(TC_ROWS)
        kp_ref[0, 0, 1, pl.ds(q * TC_ROWS, TC_ROWS)] = fy.reshape(TC_ROWS)

    return pl.pallas_call(
        body,
        grid=(k, steps),
        in_specs=[
            pl.BlockSpec((1, su, twx, 8, 128),
                         lambda kk, t: (kk, t0 * (128 // 8) // su + t,
                                        0, 0, 0)),
            pl.BlockSpec((1, su, twy, 8, 128),
                         lambda kk, t: (kk, t0 * (128 // 8) // su + t,
                                        0, 0, 0)),
        ],
        out_specs=[
            pl.BlockSpec((1, 1, 2, 128),
                         lambda kk, t: (kk, t // (128 // TC_ROWS), 0, 0)),
            pl.BlockSpec((1, ntiles, 128), lambda kk, t: (kk, 0, 0)),
        ],
        out_shape=[
            jax.ShapeDtypeStruct((k, ntiles, 2, 128), jnp.float32),
            jax.ShapeDtypeStruct((k, ntiles, 128), jnp.float32),
        ],
    )


def _tiled_view(a):
    n, k, w = a.shape
    at = a.transpose(1, 0, 2).reshape(k, n // 8, 8, w // 128, 128)
    return at.transpose(0, 1, 3, 2, 4)


def kernel(simcc_x, simcc_y):
    n, k, wx = simcc_x.shape
    wy = simcc_y.shape[-1]
    ntil = n // 128
    n_sc = T_SC * 128
    x5 = _tiled_view(simcc_x)
    y5 = _tiled_view(simcc_y)
    sc_call = _make_sc_kernel(n, k, wx, wy, n_sc)
    kp_sc, scores_sc = sc_call(x5, y5)
    tc_call = _make_tc_kernel(n, k, wx, wy, T_SC, ntil - T_SC)
    kp_tc, scores_tc = tc_call(x5, y5)
    kp_full = jnp.concatenate(
        [kp_sc.reshape(k, T_SC, 2, 128), kp_tc], axis=1)
    kp = kp_full.transpose(1, 3, 0, 2).reshape(n, k, 2)
    scores_tc_nk = scores_tc.transpose(1, 2, 0).reshape(n - n_sc, k)
    scores = jnp.concatenate([scores_sc.reshape(n_sc, k), scores_tc_nk],
                             axis=0)
    return kp, scores


# speedup vs baseline: 1.2758x; 1.2758x over previous
import functools

import jax
import jax.numpy as jnp
from jax import lax
from jax.experimental import pallas as pl
from jax.experimental.pallas import tpu as pltpu
from jax.experimental.pallas import tpu_sc as plsc

L = 16
NWORKERS = 32
NACC = 4
TN_PER_CHUNK = 2
T_SC = 12
TC_ROWS = 128


def _row_max_argmax(load, nvec, iota):
    vm = []
    vi = []
    for a in range(NACC):
        vm.append(load(a))
        vi.append(jnp.full((L,), a, jnp.int32))
    for j in range(NACC, nvec):
        a = j % NACC
        v = load(j)
        pred = v > vm[a]
        vm[a] = jnp.where(pred, v, vm[a])
        vi[a] = jnp.where(pred, j, vi[a])

    def merge(m1, i1, m2, i2):
        take = (m2 > m1) | ((m2 == m1) & (i2 < i1))
        return jnp.where(take, m2, m1), jnp.where(take, i2, i1)

    m01, i01 = merge(vm[0], vi[0], vm[1], vi[1])
    m23, i23 = merge(vm[2], vi[2], vm[3], vi[3])
    m, i = merge(m01, i01, m23, i23)

    mval = jnp.max(m)
    eidx = i * L + iota
    sel = jnp.where(m == mval, eidx, jnp.int32(2147483647))
    return mval, jnp.min(sel)


def _make_sc_kernel(n, k, wx, wy, n_sc):
    twx = wx // 128
    twy = wy // 128
    tn_per_w = (n_sc // 8) // NWORKERS
    cc_per_k = tn_per_w // TN_PER_CHUNK
    nchunks = k * cc_per_k
    nloc = n_sc // NWORKERS
    rows_per_w = nloc * k
    mesh = plsc.VectorSubcoreMesh(core_axis_name="c", subcore_axis_name="s")

    @functools.partial(
        pl.kernel,
        mesh=mesh,
        compiler_params=pltpu.CompilerParams(needs_layout_passes=False),
        out_type=[
            jax.ShapeDtypeStruct((k * n_sc * 2,), jnp.float32),
            jax.ShapeDtypeStruct((k * n_sc,), jnp.float32),
        ],
        scratch_types=[
            pltpu.VMEM((TN_PER_CHUNK, twx, 8, 128), jnp.float32),
            pltpu.VMEM((TN_PER_CHUNK, twx, 8, 128), jnp.float32),
            pltpu.VMEM((TN_PER_CHUNK, twy, 8, 128), jnp.float32),
            pltpu.VMEM((TN_PER_CHUNK, twy, 8, 128), jnp.float32),
            pltpu.VMEM((rows_per_w * 2,), jnp.float32),
            pltpu.VMEM((rows_per_w,), jnp.float32),
            pltpu.SemaphoreType.DMA,
            pltpu.SemaphoreType.DMA,
            pltpu.SemaphoreType.DMA,
            pltpu.SemaphoreType.DMA,
        ],
    )
    def sc_kernel(x_hbm, y_hbm, kp_hbm, sc_hbm,
                  xb0, xb1, yb0, yb1, kp_v, sc_v,
                  sx0, sx1, sy0, sy1):
        wid = lax.axis_index("s") * 2 + lax.axis_index("c")
        tn0 = wid * tn_per_w
        xbufs = (xb0, xb1)
        ybufs = (yb0, yb1)
        xsems = (sx0, sx1)
        ysems = (sy0, sy1)

        def srcs(g):
            kk = g // cc_per_k
            cc = g % cc_per_k
            t_lo = tn0 + cc * TN_PER_CHUNK
            return (x_hbm.at[kk, pl.ds(t_lo, TN_PER_CHUNK)],
                    y_hbm.at[kk, pl.ds(t_lo, TN_PER_CHUNK)])

        for b in range(2):
            xs, ys = srcs(b)
            pltpu.make_async_copy(xs, xbufs[b], xsems[b]).start()
            pltpu.make_async_copy(ys, ybufs[b], ysems[b]).start()

        def do_chunk(g, b):
            xs, ys = srcs(g)
            pltpu.make_async_copy(xs, xbufs[b], xsems[b]).wait()
            pltpu.make_async_copy(ys, ybufs[b], ysems[b]).wait()
            kk = g // cc_per_k
            cc = g % cc_per_k

            iota = lax.iota(jnp.int32, L)
            zero = jnp.zeros((L,), jnp.float32)

            for h in range(TN_PER_CHUNK * 8 // L):

                def row_body(rr, carry, h=h):
                    valv, fxv, fyv = carry
                    s = rr // 8
                    r = rr % 8

                    def xload(j):
                        return xbufs[b][s, j // 8, r, pl.ds((j % 8) * L, L)]

                    def yload(j):
                        return ybufs[b][s, j // 8, r, pl.ds((j % 8) * L, L)]

                    xm, xi = _row_max_argmax(xload, (twx * 128) // L, iota)
                    ym, yi = _row_max_argmax(yload, (twy * 128) // L, iota)
                    val = jnp.minimum(xm, ym)
                    neg = val <= jnp.float32(0.0)
                    fx = jnp.where(neg, jnp.float32(-1.0),
                                   xi.astype(jnp.float32)) * jnp.float32(0.5)
                    fy = jnp.where(neg, jnp.float32(-1.0),
                                   yi.astype(jnp.float32)) * jnp.float32(0.5)
                    lanehit = iota == (rr - h * L if h else rr)
                    return (jnp.where(lanehit, val, valv),
                            jnp.where(lanehit, fx, fxv),
                            jnp.where(lanehit, fy, fyv))

                valv, fxv, fyv = lax.fori_loop(
                    h * L, (h + 1) * L, row_body, (zero, zero, zero))
                m0 = TN_PER_CHUNK * 8 * cc + h * L
                plsc.store_scatter(sc_v, [(m0 * k + kk) + k * iota], valv)
                kp_v[pl.ds(kk * 2 * nloc + m0, L)] = fxv
                kp_v[pl.ds(kk * 2 * nloc + nloc + m0, L)] = fyv

            @pl.when(g + 2 < nchunks)
            def _():
                xs2, ys2 = srcs(g + 2)
                pltpu.make_async_copy(xs2, xbufs[b], xsems[b]).start()
                pltpu.make_async_copy(ys2, ybufs[b], ysems[b]).start()

        def pair_body(i, _):
            do_chunk(2 * i, 0)
            do_chunk(2 * i + 1, 1)
            return 0

        lax.fori_loop(0, nchunks // 2, pair_body, 0)
        if nchunks % 2:
            do_chunk(nchunks - 1, 0)

        kp_copies = []
        for kk_s in range(k):
            for c in range(2):
                src = kp_v.at[pl.ds(kk_s * 2 * nloc + c * nloc, nloc)]
                dst = kp_hbm.at[pl.ds(kk_s * 2 * n_sc
                                      + (wid * nloc // 128) * 256
                                      + c * 128 + (wid * nloc) % 128, nloc)]
                kp_copies.append(pltpu.make_async_copy(src, dst, sx0))
        for cp in kp_copies:
            cp.start()
        for cp in kp_copies:
            cp.wait()
        pltpu.sync_copy(sc_v, sc_hbm.at[pl.ds(wid * rows_per_w, rows_per_w)])

    return sc_kernel


def _make_tc_kernel(n, k, wx, wy, t0, ntiles):
    twx = wx // 128
    twy = wy // 128
    n_tc = ntiles * 128
    steps = n_tc // TC_ROWS
    su = TC_ROWS // 8

    def body(x_ref, y_ref, kp_ref, sc_ref):
        big = jnp.int32(2147483647)
        q = lax.rem(pl.program_id(1), 128 // TC_ROWS)

        def stats(b):
            shape = b.shape
            ei = (lax.broadcasted_iota(jnp.int32, shape, 1) * 128
                  + lax.broadcasted_iota(jnp.int32, shape, 3))
            m = jnp.max(b, axis=(1, 3))
            sel = jnp.where(b == m[:, None, :, None], ei, big)
            return m, jnp.min(sel, axis=(1, 3))

        xm, xi = stats(x_ref[0])
        ym, yi = stats(y_ref[0])
        val = jnp.minimum(xm, ym)
        neg = val <= jnp.float32(0.0)
        fx = jnp.where(neg, jnp.float32(-1.0),
                       xi.astype(jnp.float32)) * jnp.float32(0.5)
        fy = jnp.where(neg, jnp.float32(-1.0),
                       yi.astype(jnp.float32)) * jnp.float32(0.5)
        tt = pl.program_id(1) // (128 // TC_ROWS)
        sc_ref[0, tt, pl.ds(q * TC_ROWS, TC_ROWS)] = val.reshape(TC_ROWS)
        kp_ref[0, 0, 0, pl.ds(q * TC_ROWS, TC_ROWS)] = fx.reshape(TC_ROWS)
        kp_ref[0, 0, 1, pl.ds(q * TC_ROWS, TC_ROWS)] = fy.reshape(TC_ROWS)

    return pl.pallas_call(
        body,
        grid=(k, steps),
        in_specs=[
            pl.BlockSpec((1, su, twx, 8, 128),
                         lambda kk, t: (kk, t0 * (128 // 8) // su + t,
                                        0, 0, 0)),
            pl.BlockSpec((1, su, twy, 8, 128),
                         lambda kk, t: (kk, t0 * (128 // 8) // su + t,
                                        0, 0, 0)),
        ],
        out_specs=[
            pl.BlockSpec((1, 1, 2, 128),
                         lambda kk, t: (kk, t // (128 // TC_ROWS), 0, 0)),
            pl.BlockSpec((1, ntiles, 128), lambda kk, t: (kk, 0, 0)),
        ],
        out_shape=[
            jax.ShapeDtypeStruct((k, ntiles, 2, 128), jnp.float32),
            jax.ShapeDtypeStruct((k, ntiles, 128), jnp.float32),
        ],
    )


def _tiled_view(a):
    n, k, w = a.shape
    at = a.transpose(1, 0, 2).reshape(k, n // 8, 8, w // 128, 128)
    return at.transpose(0, 1, 3, 2, 4)


def kernel(simcc_x, simcc_y):
    n, k, wx = simcc_x.shape
    wy = simcc_y.shape[-1]
    ntil = n // 128
    n_sc = T_SC * 128
    x5 = _tiled_view(simcc_x)
    y5 = _tiled_view(simcc_y)
    sc_call = _make_sc_kernel(n, k, wx, wy, n_sc)
    kp_sc, scores_sc = sc_call(x5, y5)
    tc_call = _make_tc_kernel(n, k, wx, wy, T_SC, ntil - T_SC)
    kp_tc, scores_tc = tc_call(x5, y5)
    kp_full = jnp.concatenate(
        [kp_sc.reshape(k, T_SC, 2, 128), kp_tc], axis=1)
    kp = kp_full.transpose(1, 3, 0, 2).reshape(n, k, 2)
    scores_tc_nk = scores_tc.transpose(1, 2, 0).reshape(n - n_sc, k)
    scores = jnp.concatenate([scores_sc.reshape(n_sc, k), scores_tc_nk],
                             axis=0)
    return kp, scores
